# baseline (speedup 1.0000x reference)
import jax
import jax.numpy as jnp
from jax.experimental import pallas as pl
from jax.experimental.pallas import tpu as pltpu

_IN = 6
_HID = 32
_OUT = 2
_TILE_L = 32768


def _mlp_kernel(x_ref, s_ref, o_ref):
    w1t = s_ref[0:_IN, 0:_HID] * 0.5
    h = jax.lax.dot_general(w1t, x_ref[...],
                            (((0,), (0,)), ((), ())),
                            preferred_element_type=jnp.float32)
    b1c = jnp.transpose(s_ref[8:9, 0:_HID]) * 0.5
    t = jnp.tanh(h + b1c)
    w2t = s_ref[9:9 + _OUT, 0:_HID]
    o = jnp.dot(w2t * 0.25, t,
                preferred_element_type=jnp.float32)
    b2c = (jnp.transpose(s_ref[11:12, 0:_OUT])
           + 0.5 * jnp.sum(w2t, axis=1, keepdims=True)) * 0.5
    o_ref[...] = 0.5 * jnp.tanh(o + b2c) + 0.5


def kernel(x_batch, param_slab):
    B = x_batch.shape[0]
    b_pad = pl.cdiv(B, _TILE_L) * _TILE_L
    if b_pad != B:
        x_batch = jnp.pad(x_batch.astype(jnp.float32),
                          ((0, b_pad - B), (0, 0)))
    xt = x_batch.T

    n_tiles = b_pad // _TILE_L
    out = pl.pallas_call(
        _mlp_kernel,
        out_shape=jax.ShapeDtypeStruct((_OUT, b_pad), jnp.float32),
        grid=(n_tiles,),
        in_specs=[
            pl.BlockSpec((_IN, _TILE_L), lambda i: (0, i)),
            pl.BlockSpec((16, 128), lambda i: (0, 0)),
        ],
        out_specs=pl.BlockSpec((_OUT, _TILE_L), lambda i: (0, i)),
        compiler_params=pltpu.CompilerParams(
            dimension_semantics=("parallel",)),
    )(xt, param_slab)
    return out[:, :B].T


# speedup vs baseline: 1.3120x; 1.3120x over previous
import jax
import jax.numpy as jnp
from jax.experimental import pallas as pl
from jax.experimental.pallas import tpu as pltpu

_IN = 6
_HID = 32
_OUT = 2
_TILE_L = 131072


def _mlp_kernel(x_ref, s_ref, o_ref):
    w1t = s_ref[0:_IN, 0:_HID] * 0.5
    h = jax.lax.dot_general(w1t, x_ref[...],
                            (((0,), (0,)), ((), ())),
                            preferred_element_type=jnp.float32)
    b1c = jnp.transpose(s_ref[8:9, 0:_HID]) * 0.5
    t = jnp.tanh(h + b1c)
    w2t = s_ref[9:9 + _OUT, 0:_HID]
    o = jnp.dot(w2t * 0.25, t,
                preferred_element_type=jnp.float32)
    b2c = (jnp.transpose(s_ref[11:12, 0:_OUT])
           + 0.5 * jnp.sum(w2t, axis=1, keepdims=True)) * 0.5
    o_ref[...] = 0.5 * jnp.tanh(o + b2c) + 0.5


def kernel(x_batch, param_slab):
    B = x_batch.shape[0]
    b_pad = pl.cdiv(B, _TILE_L) * _TILE_L
    if b_pad != B:
        x_batch = jnp.pad(x_batch.astype(jnp.float32),
                          ((0, b_pad - B), (0, 0)))
    xt = x_batch.T

    n_tiles = b_pad // _TILE_L
    out = pl.pallas_call(
        _mlp_kernel,
        out_shape=jax.ShapeDtypeStruct((_OUT, b_pad), jnp.float32),
        grid=(n_tiles,),
        in_specs=[
            pl.BlockSpec((_IN, _TILE_L), lambda i: (0, i)),
            pl.BlockSpec((16, 128), lambda i: (0, 0)),
        ],
        out_specs=pl.BlockSpec((_OUT, _TILE_L), lambda i: (0, i)),
        compiler_params=pltpu.CompilerParams(
            dimension_semantics=("parallel",)),
    )(xt, param_slab)
    return out[:, :B].T
